# Initial kernel scaffold; baseline (speedup 1.0000x reference)
import jax
import jax.numpy as jnp
from jax.experimental import pallas as pl


def kernel(edge_index, edge_features, params):
    raise NotImplementedError("write your pallas kernel here")


# speedup vs baseline: 6.5595x; 6.5595x over previous
import functools

import jax
import jax.numpy as jnp
from jax import lax
from jax.experimental import pallas as pl
from jax.experimental.pallas import tpu as pltpu
from jax.experimental.pallas import tpu_sc as plsc

_N = 10000
_NC, _NS = 2, 16
_NW = _NC * _NS
_CH = 80

_F32 = jnp.float32


def _ln_rows(x, s, b):
    m = jnp.mean(x, axis=-1, keepdims=True)
    v = jnp.mean((x - m) ** 2, axis=-1, keepdims=True)
    return (x - m) * lax.rsqrt(v + 1e-6) * s + b


def _dot(a, b):
    return jnp.dot(a, b, preferred_element_type=_F32)


@functools.lru_cache(maxsize=None)
def _make_sc_gather(n_idx, w):
    per_w = n_idx // _NW
    n_ch = per_w // _CH
    assert per_w * _NW == n_idx and n_ch * _CH == per_w
    mesh = plsc.VectorSubcoreMesh(core_axis_name="c", subcore_axis_name="s")

    @functools.partial(
        pl.kernel,
        out_type=(jax.ShapeDtypeStruct((n_idx, w), _F32),
                  jax.ShapeDtypeStruct((n_idx, w), _F32)),
        mesh=mesh,
        scratch_types=[pltpu.VMEM((_CH,), jnp.int32),
                       pltpu.VMEM((_CH, w), _F32),
                       pltpu.SemaphoreType.DMA],
        compiler_params=pltpu.CompilerParams(use_tc_tiling_on_sc=False),
    )
    def k(table, idx1, idx2, out1, out2, idx_v, rows_v, sem):
        c = lax.axis_index("c")
        s = lax.axis_index("s")
        wid = s * _NC + c
        base0 = wid * per_w

        def make_body(idx_hbm, out_hbm):
            def body(i, carry):
                base = base0 + i * _CH
                pltpu.sync_copy(idx_hbm.at[pl.ds(base, _CH)], idx_v)
                pltpu.async_copy(table.at[idx_v], rows_v, sem).wait()
                pltpu.sync_copy(rows_v, out_hbm.at[pl.ds(base, _CH)])
                return carry
            return body

        lax.fori_loop(0, n_ch, make_body(idx1, out1), 0)
        lax.fori_loop(0, n_ch, make_body(idx2, out2), 0)

    return k


@functools.lru_cache(maxsize=None)
def _make_sc_scatter(n_idx, n_rows, w):
    half = n_rows // _NC
    n_ch = n_idx // _CH // _NS
    assert n_ch * _CH * _NS == n_idx and half * _NC == n_rows
    assert half % 8 == 0
    m8 = half // 8
    q8 = -(-(m8 + 1) // _NS)
    mesh = plsc.VectorSubcoreMesh(core_axis_name="c", subcore_axis_name="s")

    @functools.partial(
        pl.kernel,
        out_type=jax.ShapeDtypeStruct((n_rows, w), _F32),
        mesh=mesh,
        scratch_types=[pltpu.VMEM((_CH,), jnp.int32),
                       pltpu.VMEM((_CH, w), _F32),
                       pltpu.VMEM((8, w), _F32),
                       pltpu.VMEM_SHARED((half + 8, w), _F32),
                       pltpu.SemaphoreType.DMA],
        compiler_params=pltpu.CompilerParams(use_tc_tiling_on_sc=False),
    )
    def k(idx1, idx2, vals1, vals2, zeros_hbm, out,
          idx_v, val_v, z_v, acc, sem):
        c = lax.axis_index("c")
        s = lax.axis_index("s")
        row_base = c * half

        pltpu.sync_copy(zeros_hbm.at[pl.ds(0, 8)], z_v)

        def zbody(i, carry):
            ch = s * q8 + i
            @pl.when(ch < m8 + 1)
            def _():
                pltpu.sync_copy(z_v, acc.at[pl.ds(ch * 8, 8)])
            return carry
        lax.fori_loop(0, q8, zbody, 0)
        plsc.subcore_barrier()

        def make_body(idx_hbm, vals_hbm):
            def body(i, carry):
                base = (s * n_ch + i) * _CH
                pltpu.sync_copy(idx_hbm.at[pl.ds(c * n_idx + base, _CH)],
                                idx_v)
                pltpu.sync_copy(vals_hbm.at[pl.ds(base, _CH)], val_v)
                pltpu.sync_copy(val_v, acc.at[idx_v], add=True)
                return carry
            return body

        lax.fori_loop(0, n_ch, make_body(idx1, vals1), 0)
        lax.fori_loop(0, n_ch, make_body(idx2, vals2), 0)
        plsc.subcore_barrier()

        def wbody(i, carry):
            ch = s * q8 + i
            @pl.when(ch < m8)
            def _():
                pltpu.sync_copy(acc.at[pl.ds(ch * 8, 8)],
                                out.at[pl.ds(row_base + ch * 8, 8)])
            return carry
        lax.fori_loop(0, q8, wbody, 0)

    return k


@functools.lru_cache(maxsize=None)
def _make_sc_counts(n_idx, n_rows):
    w = 16
    half = n_rows // _NC
    n_ch = n_idx // _CH // _NS
    assert n_ch * _CH * _NS == n_idx and half * _NC == n_rows
    assert half % 8 == 0
    m8 = half // 8
    q8 = -(-(m8 + 1) // _NS)
    mesh = plsc.VectorSubcoreMesh(core_axis_name="c", subcore_axis_name="s")

    @functools.partial(
        pl.kernel,
        out_type=jax.ShapeDtypeStruct((n_rows, w), _F32),
        mesh=mesh,
        scratch_types=[pltpu.VMEM((_CH,), jnp.int32),
                       pltpu.VMEM((_CH, w), _F32),
                       pltpu.VMEM((8, w), _F32),
                       pltpu.VMEM_SHARED((half + 8, w), _F32),
                       pltpu.SemaphoreType.DMA],
        compiler_params=pltpu.CompilerParams(use_tc_tiling_on_sc=False),
    )
    def k(idx_hbm, ones_hbm, zeros_hbm, out, idx_v, ones_v, z_v, acc, sem):
        c = lax.axis_index("c")
        s = lax.axis_index("s")
        row_base = c * half

        pltpu.sync_copy(zeros_hbm.at[pl.ds(0, 8)], z_v)
        pltpu.sync_copy(ones_hbm, ones_v)

        def zbody(i, carry):
            ch = s * q8 + i
            @pl.when(ch < m8 + 1)
            def _():
                pltpu.sync_copy(z_v, acc.at[pl.ds(ch * 8, 8)])
            return carry
        lax.fori_loop(0, q8, zbody, 0)
        plsc.subcore_barrier()

        def body(i, carry):
            base = (s * n_ch + i) * _CH
            pltpu.sync_copy(idx_hbm.at[pl.ds(c * n_idx + base, _CH)], idx_v)
            pltpu.sync_copy(ones_v, acc.at[idx_v], add=True)
            return carry
        lax.fori_loop(0, n_ch, body, 0)
        plsc.subcore_barrier()

        def wbody(i, carry):
            ch = s * q8 + i
            @pl.when(ch < m8)
            def _():
                pltpu.sync_copy(acc.at[pl.ds(ch * 8, 8)],
                                out.at[pl.ds(row_base + ch * 8, 8)])
            return carry
        lax.fori_loop(0, q8, wbody, 0)

    return k


def _sc_gather(table, idx1, idx2):
    n_idx = idx1.shape[0]
    return _make_sc_gather(n_idx, table.shape[1])(table, idx1, idx2)


def _localize(idx, n_rows):
    half = n_rows // _NC
    rows = []
    for c in range(_NC):
        l = idx - c * half
        rows.append(jnp.where((l >= 0) & (l < half), l, half))
    return jnp.concatenate(rows)


def _sc_scatter(idx1, idx2, vals1, vals2, n_rows):
    w = vals1.shape[1]
    return _make_sc_scatter(idx1.shape[0], n_rows, w)(
        _localize(idx1, n_rows), _localize(idx2, n_rows),
        vals1, vals2, jnp.zeros((8, w), _F32))


def _sc_counts(idx, n_rows):
    return _make_sc_counts(idx.shape[0], n_rows)(
        _localize(idx, n_rows), jnp.ones((_CH, 16), _F32),
        jnp.zeros((8, 16), _F32))


def _row_block(n_rows):
    for r in (640, 512, 320, 160, 80, 8):
        if n_rows % r == 0:
            return r
    return n_rows


def _full(shape):
    return pl.BlockSpec(shape, lambda *_: tuple(0 for _ in shape))


def _tc_embed(x, wee, bee, wm2, bne, wm1, bm):
    be_rows = x.shape[0]
    r = _row_block(be_rows)
    h = wee.shape[1]

    def body(x_ref, wee_ref, bee_ref, wm2_ref, bne_ref, wm1_ref, bm_ref,
             ef_ref, msg_ref):
        ef = _dot(x_ref[...], wee_ref[...]) + bee_ref[...]
        ef_ref[...] = ef
        c0 = _dot(bne_ref[...], wm1_ref[...]) + bm_ref[...]
        msg_ref[...] = jnp.maximum(_dot(ef, wm2_ref[...]) + c0, 0.0)

    return pl.pallas_call(
        body,
        grid=(be_rows // r,),
        in_specs=[pl.BlockSpec((r, x.shape[1]), lambda i: (i, 0)),
                  _full(wee.shape), _full(bee.shape), _full(wm2.shape),
                  _full(bne.shape), _full(wm1.shape), _full(bm.shape)],
        out_specs=[pl.BlockSpec((r, h), lambda i: (i, 0)),
                   pl.BlockSpec((r, h), lambda i: (i, 0))],
        out_shape=[jax.ShapeDtypeStruct((be_rows, h), _F32),
                   jax.ShapeDtypeStruct((be_rows, h), _F32)],
    )(x, wee, bee, wm2, bne, wm1, bm)


def _tc_node_update(agg, counts, prev, s, b):
    bn_rows, h = prev.shape
    r = _row_block(bn_rows)

    def body(agg_ref, cnt_ref, prev_ref, s_ref, b_ref, out_ref):
        cnt = jnp.maximum(cnt_ref[:, 0:1], 1.0)
        out_ref[...] = _ln_rows(agg_ref[...] / cnt + prev_ref[...],
                                s_ref[...], b_ref[...])

    rb = lambda i: (i, 0)
    return pl.pallas_call(
        body,
        grid=(bn_rows // r,),
        in_specs=[pl.BlockSpec((r, h), rb),
                  pl.BlockSpec((r, counts.shape[1]), lambda i: (i, 0)),
                  pl.BlockSpec((r, h), rb), _full(s.shape), _full(b.shape)],
        out_specs=pl.BlockSpec((r, h), rb),
        out_shape=jax.ShapeDtypeStruct((bn_rows, h), _F32),
    )(agg, counts, prev, s, b)


def _tc_edge_block(nr, nc, ef, wn1, wn2, bn, we, be, wc1, wc2, bc, s2, b2,
                   wm1, wm2, bm):
    be_rows, h = ef.shape
    r = _row_block(be_rows)

    def body(nr_ref, nc_ref, ef_ref, wn1_ref, wn2_ref, bn_ref, we_ref,
             be_ref, wc1_ref, wc2_ref, bc_ref, s2_ref, b2_ref,
             wm1_ref, wm2_ref, bm_ref, ef_out, m1_out, m2_out):
        nr_ = nr_ref[...]
        nc_ = nc_ref[...]
        ef_ = ef_ref[...]
        np1 = jnp.maximum(_dot(nr_, wn1_ref[...]) + _dot(nc_, wn2_ref[...])
                          + bn_ref[...], 0.0)
        np2 = jnp.maximum(_dot(nc_, wn1_ref[...]) + _dot(nr_, wn2_ref[...])
                          + bn_ref[...], 0.0)
        ep = jnp.maximum(_dot(ef_, we_ref[...]) + be_ref[...], 0.0)
        sc = _dot(ep, wc2_ref[...]) + bc_ref[...]
        o1 = jnp.maximum(_dot(np1, wc1_ref[...]) + sc, 0.0)
        o2 = jnp.maximum(_dot(np2, wc1_ref[...]) + sc, 0.0)
        efn = _ln_rows((o1 + o2) * 0.5 + ef_, s2_ref[...], b2_ref[...])
        ef_out[...] = efn
        t = _dot(efn, wm2_ref[...]) + bm_ref[...]
        m1_out[...] = jnp.maximum(_dot(nr_, wm1_ref[...]) + t, 0.0)
        m2_out[...] = jnp.maximum(_dot(nc_, wm1_ref[...]) + t, 0.0)

    rb = lambda i: (i, 0)
    w = [wn1, wn2, bn, we, be, wc1, wc2, bc, s2, b2, wm1, wm2, bm]
    return pl.pallas_call(
        body,
        grid=(be_rows // r,),
        in_specs=[pl.BlockSpec((r, h), rb)] * 3 + [_full(a.shape) for a in w],
        out_specs=[pl.BlockSpec((r, h), rb)] * 3,
        out_shape=[jax.ShapeDtypeStruct((be_rows, h), _F32)] * 3,
    )(nr, nc, ef, *w)


def _tc_edge_final(nr, nc, ef, wn1, wn2, bn, we, be, wc1, wc2, bc, s2, b2,
                   wp1, bp1, wp2, bp2, wp3, bp3):
    be_rows, h = ef.shape
    r = _row_block(be_rows)

    def body(nr_ref, nc_ref, ef_ref, wn1_ref, wn2_ref, bn_ref, we_ref,
             be_ref, wc1_ref, wc2_ref, bc_ref, s2_ref, b2_ref,
             wp1_ref, bp1_ref, wp2_ref, bp2_ref, wp3_ref, bp3_ref, out_ref):
        nr_ = nr_ref[...]
        nc_ = nc_ref[...]
        ef_ = ef_ref[...]
        np1 = jnp.maximum(_dot(nr_, wn1_ref[...]) + _dot(nc_, wn2_ref[...])
                          + bn_ref[...], 0.0)
        np2 = jnp.maximum(_dot(nc_, wn1_ref[...]) + _dot(nr_, wn2_ref[...])
                          + bn_ref[...], 0.0)
        ep = jnp.maximum(_dot(ef_, we_ref[...]) + be_ref[...], 0.0)
        sc = _dot(ep, wc2_ref[...]) + bc_ref[...]
        o1 = jnp.maximum(_dot(np1, wc1_ref[...]) + sc, 0.0)
        o2 = jnp.maximum(_dot(np2, wc1_ref[...]) + sc, 0.0)
        efn = _ln_rows((o1 + o2) * 0.5 + ef_, s2_ref[...], b2_ref[...])
        x1 = jnp.maximum(_dot(efn, wp1_ref[...]) + bp1_ref[...], 0.0)
        x2 = jnp.maximum(_dot(x1, wp2_ref[...]) + bp2_ref[...], 0.0)
        out_ref[...] = _dot(x2, wp3_ref[...]) + bp3_ref[...]

    rb = lambda i: (i, 0)
    w = [wn1, wn2, bn, we, be, wc1, wc2, bc, s2, b2,
         wp1, bp1, wp2, bp2, wp3, bp3]
    return pl.pallas_call(
        body,
        grid=(be_rows // r,),
        in_specs=[pl.BlockSpec((r, h), rb)] * 3 + [_full(a.shape) for a in w],
        out_specs=pl.BlockSpec((r, 1), rb),
        out_shape=jax.ShapeDtypeStruct((be_rows, 1), _F32),
    )(nr, nc, ef, *w)


def _tc_softmax(logits3):
    bsz, rows, lanes = logits3.shape

    def body(x_ref, out_ref):
        x = x_ref[...]
        m = jnp.max(x)
        e = jnp.exp(x - m)
        out_ref[...] = e / jnp.sum(e)

    return pl.pallas_call(
        body,
        grid=(bsz,),
        in_specs=[pl.BlockSpec((1, rows, lanes), lambda i: (i, 0, 0))],
        out_specs=pl.BlockSpec((1, rows, lanes), lambda i: (i, 0, 0)),
        out_shape=jax.ShapeDtypeStruct(logits3.shape, _F32),
    )(logits3)


def _tc_value_head(nodes, bsz, n, wv1, bv1, wv2, bv2):
    h = nodes.shape[1]

    def body(nodes_ref, wv1_ref, bv1_ref, wv2_ref, bv2_ref, out_ref):
        g = jnp.mean(nodes_ref[...].reshape(bsz, n, h), axis=1)
        v = jnp.maximum(_dot(g, wv1_ref[...]) + bv1_ref[...], 0.0)
        out_ref[...] = jnp.broadcast_to(
            jnp.tanh(_dot(v, wv2_ref[...]) + bv2_ref[...]), (bsz, 128))

    out = pl.pallas_call(
        body,
        in_specs=[_full(nodes.shape),
                  _full(wv1.shape), _full(bv1.shape),
                  _full(wv2.shape), _full(bv2.shape)],
        out_specs=_full((bsz, 128)),
        out_shape=jax.ShapeDtypeStruct((bsz, 128), _F32),
    )(nodes, wv1, bv1, wv2, bv2)
    return out[:, :1]


def kernel(edge_index, edge_features, params):
    bsz, _, e = edge_index.shape
    n = _N
    be = bsz * e
    bn_rows = bsz * n
    h = params['W_ee'].shape[1]
    nlayers = params['Wm'].shape[0]

    row = edge_index[:, 0, :]
    col = edge_index[:, 1, :]
    offs = (jnp.arange(bsz, dtype=jnp.int32) * n)[:, None]
    rg = (row + offs).reshape(-1)
    cg = (col + offs).reshape(-1)
    idx_all = jnp.concatenate([cg, rg])

    r1 = lambda a: a.reshape(1, -1)

    counts = _sc_counts(idx_all, bn_rows)

    x = jnp.pad(edge_features.reshape(be, edge_features.shape[2]),
                ((0, 0), (0, 5)))
    wee = jnp.pad(params['W_ee'], ((0, 5), (0, 0)))
    ef, msg0 = _tc_embed(x, wee, r1(params['b_ee']),
                         params['Wm'][0][h:], r1(params['b_ne']),
                         params['Wm'][0][:h], r1(params['bm'][0]))

    prev = jnp.broadcast_to(r1(params['b_ne']), (bn_rows, h))
    m1, m2 = msg0, msg0
    nodes = None
    for l in range(nlayers):
        agg = _sc_scatter(cg, rg, m1, m2, bn_rows)
        nodes = _tc_node_update(agg, counts, prev,
                                r1(params['ln1_s'][l]), r1(params['ln1_b'][l]))
        nr, nc = _sc_gather(nodes, rg, cg)
        wn, wc = params['Wn'][l], params['Wc'][l]
        if l < nlayers - 1:
            ef, m1, m2 = _tc_edge_block(
                nr, nc, ef, wn[:h], wn[h:], r1(params['bn'][l]),
                params['We'][l], r1(params['be'][l]),
                wc[:h], wc[h:], r1(params['bc'][l]),
                r1(params['ln2_s'][l]), r1(params['ln2_b'][l]),
                params['Wm'][l + 1][:h], params['Wm'][l + 1][h:],
                r1(params['bm'][l + 1]))
            prev = nodes
        else:
            logits = _tc_edge_final(
                nr, nc, ef, wn[:h], wn[h:], r1(params['bn'][l]),
                params['We'][l], r1(params['be'][l]),
                wc[:h], wc[h:], r1(params['bc'][l]),
                r1(params['ln2_s'][l]), r1(params['ln2_b'][l]),
                params['Wp1'], r1(params['bp1']), params['Wp2'],
                r1(params['bp2']), params['Wp3'], r1(params['bp3']))

    lanes = 128
    policies = _tc_softmax(logits.reshape(bsz, e // lanes, lanes))
    policies = policies.reshape(bsz, e)
    values = _tc_value_head(nodes, bsz, n, params['Wv1'], r1(params['bv1']),
                            params['Wv2'], r1(params['bv2']))
    return policies, values


# speedup vs baseline: 7.8622x; 1.1986x over previous
import functools

import jax
import jax.numpy as jnp
from jax import lax
from jax.experimental import pallas as pl
from jax.experimental.pallas import tpu as pltpu
from jax.experimental.pallas import tpu_sc as plsc

_N = 10000
_NC, _NS = 2, 16
_NW = _NC * _NS
_CH = 80

_F32 = jnp.float32


def _ln_rows(x, s, b):
    m = jnp.mean(x, axis=-1, keepdims=True)
    v = jnp.mean((x - m) ** 2, axis=-1, keepdims=True)
    return (x - m) * lax.rsqrt(v + 1e-6) * s + b


def _dot(a, b):
    return jnp.dot(a, b, preferred_element_type=_F32)


def _nbuf_for(n_ch):
    for nb in (2, 5, 3):
        if n_ch % nb == 0:
            return nb
    return 1


@functools.lru_cache(maxsize=None)
def _make_sc_gather(n_idx, w):
    per_w = n_idx // _NW
    n_ch = per_w // _CH
    assert per_w * _NW == n_idx and n_ch * _CH == per_w
    nbuf = _nbuf_for(n_ch)
    mesh = plsc.VectorSubcoreMesh(core_axis_name="c", subcore_axis_name="s")

    @functools.partial(
        pl.kernel,
        out_type=(jax.ShapeDtypeStruct((n_idx, w), _F32),
                  jax.ShapeDtypeStruct((n_idx, w), _F32)),
        mesh=mesh,
        scratch_types=[pltpu.VMEM((nbuf, _CH), jnp.int32),
                       pltpu.VMEM((nbuf, _CH, w), _F32),
                       pltpu.SemaphoreType.DMA((nbuf,)),
                       pltpu.SemaphoreType.DMA,
                       pltpu.SemaphoreType.DMA((nbuf,))],
        compiler_params=pltpu.CompilerParams(use_tc_tiling_on_sc=False),
    )
    def k(table, idx1, idx2, out1, out2, idx_v, rows_v, isem, gsem, wsem):
        c = lax.axis_index("c")
        s = lax.axis_index("s")
        wid = s * _NC + c
        base0 = wid * per_w

        def make_body(idx_hbm, out_hbm):
            pltpu.async_copy(idx_hbm.at[pl.ds(base0, _CH)],
                             idx_v.at[0], isem.at[0])

            def group(g, carry):
                for b in range(nbuf):
                    i = g * nbuf + b
                    nxt = (b + 1) % nbuf

                    @pl.when(i + 1 < n_ch)
                    def _():
                        pltpu.async_copy(
                            idx_hbm.at[pl.ds(base0 + (i + 1) * _CH, _CH)],
                            idx_v.at[nxt], isem.at[nxt])

                    pltpu.make_async_copy(
                        idx_hbm.at[pl.ds(base0, _CH)], idx_v.at[b],
                        isem.at[b]).wait()

                    @pl.when(i >= nbuf)
                    def _():
                        pltpu.make_async_copy(
                            rows_v.at[b], out_hbm.at[pl.ds(base0, _CH)],
                            wsem.at[b]).wait()

                    pltpu.async_copy(table.at[idx_v.at[b]], rows_v.at[b],
                                     gsem).wait()
                    pltpu.async_copy(rows_v.at[b],
                                     out_hbm.at[pl.ds(base0 + i * _CH, _CH)],
                                     wsem.at[b])
                return carry

            lax.fori_loop(0, n_ch // nbuf, group, 0)
            for b in range(min(nbuf, n_ch)):
                pltpu.make_async_copy(
                    rows_v.at[b], out_hbm.at[pl.ds(base0, _CH)],
                    wsem.at[b]).wait()

        make_body(idx1, out1)
        make_body(idx2, out2)

    return k


@functools.lru_cache(maxsize=None)
def _make_sc_scatter(n_idx, n_rows, w):
    half = n_rows // _NC
    n_ch = n_idx // _CH // _NS
    assert n_ch * _CH * _NS == n_idx and half * _NC == n_rows
    assert half % 8 == 0
    m8 = half // 8
    q8 = -(-(m8 + 1) // _NS)
    mesh = plsc.VectorSubcoreMesh(core_axis_name="c", subcore_axis_name="s")

    nbuf = _nbuf_for(n_ch)

    @functools.partial(
        pl.kernel,
        out_type=jax.ShapeDtypeStruct((n_rows, w), _F32),
        mesh=mesh,
        scratch_types=[pltpu.VMEM((nbuf, _CH), jnp.int32),
                       pltpu.VMEM((nbuf, _CH, w), _F32),
                       pltpu.VMEM((8, w), _F32),
                       pltpu.VMEM_SHARED((half + 8, w), _F32),
                       pltpu.SemaphoreType.DMA((nbuf,)),
                       pltpu.SemaphoreType.DMA((nbuf,))],
        compiler_params=pltpu.CompilerParams(use_tc_tiling_on_sc=False),
    )
    def k(idx1, idx2, vals1, vals2, zeros_hbm, out,
          idx_v, val_v, z_v, acc, isem, vsem):
        c = lax.axis_index("c")
        s = lax.axis_index("s")
        row_base = c * half

        pltpu.sync_copy(zeros_hbm.at[pl.ds(0, 8)], z_v)

        def zbody(i, carry):
            ch = s * q8 + i
            @pl.when(ch < m8 + 1)
            def _():
                pltpu.sync_copy(z_v, acc.at[pl.ds(ch * 8, 8)])
            return carry
        lax.fori_loop(0, q8, zbody, 0)
        plsc.subcore_barrier()

        def make_body(idx_hbm, vals_hbm):
            def load(i, b):
                base = (s * n_ch + i) * _CH
                pltpu.async_copy(idx_hbm.at[pl.ds(c * n_idx + base, _CH)],
                                 idx_v.at[b], isem.at[b])
                pltpu.async_copy(vals_hbm.at[pl.ds(base, _CH)],
                                 val_v.at[b], vsem.at[b])

            load(0, 0)

            def group(g, carry):
                for b in range(nbuf):
                    i = g * nbuf + b
                    nxt = (b + 1) % nbuf

                    @pl.when(i + 1 < n_ch)
                    def _():
                        load(i + 1, nxt)

                    pltpu.make_async_copy(
                        idx_hbm.at[pl.ds(0, _CH)], idx_v.at[b],
                        isem.at[b]).wait()
                    pltpu.make_async_copy(
                        vals_hbm.at[pl.ds(0, _CH)], val_v.at[b],
                        vsem.at[b]).wait()
                    pltpu.sync_copy(val_v.at[b], acc.at[idx_v.at[b]],
                                    add=True)
                return carry

            lax.fori_loop(0, n_ch // nbuf, group, 0)

        make_body(idx1, vals1)
        make_body(idx2, vals2)
        plsc.subcore_barrier()

        def wbody(i, carry):
            ch = s * q8 + i
            @pl.when(ch < m8)
            def _():
                pltpu.sync_copy(acc.at[pl.ds(ch * 8, 8)],
                                out.at[pl.ds(row_base + ch * 8, 8)])
            return carry
        lax.fori_loop(0, q8, wbody, 0)

    return k


@functools.lru_cache(maxsize=None)
def _make_sc_counts(n_idx, n_rows):
    w = 16
    half = n_rows // _NC
    n_ch = n_idx // _CH // _NS
    assert n_ch * _CH * _NS == n_idx and half * _NC == n_rows
    assert half % 8 == 0
    m8 = half // 8
    q8 = -(-(m8 + 1) // _NS)
    mesh = plsc.VectorSubcoreMesh(core_axis_name="c", subcore_axis_name="s")

    nbuf = _nbuf_for(n_ch)

    @functools.partial(
        pl.kernel,
        out_type=jax.ShapeDtypeStruct((n_rows, w), _F32),
        mesh=mesh,
        scratch_types=[pltpu.VMEM((nbuf, _CH), jnp.int32),
                       pltpu.VMEM((_CH, w), _F32),
                       pltpu.VMEM((8, w), _F32),
                       pltpu.VMEM_SHARED((half + 8, w), _F32),
                       pltpu.SemaphoreType.DMA((nbuf,))],
        compiler_params=pltpu.CompilerParams(use_tc_tiling_on_sc=False),
    )
    def k(idx_hbm, ones_hbm, zeros_hbm, out, idx_v, ones_v, z_v, acc, isem):
        c = lax.axis_index("c")
        s = lax.axis_index("s")
        row_base = c * half

        pltpu.sync_copy(zeros_hbm.at[pl.ds(0, 8)], z_v)
        pltpu.sync_copy(ones_hbm, ones_v)

        def zbody(i, carry):
            ch = s * q8 + i
            @pl.when(ch < m8 + 1)
            def _():
                pltpu.sync_copy(z_v, acc.at[pl.ds(ch * 8, 8)])
            return carry
        lax.fori_loop(0, q8, zbody, 0)
        plsc.subcore_barrier()

        def load(i, b):
            base = (s * n_ch + i) * _CH
            pltpu.async_copy(idx_hbm.at[pl.ds(c * n_idx + base, _CH)],
                             idx_v.at[b], isem.at[b])

        load(0, 0)

        def group(g, carry):
            for b in range(nbuf):
                i = g * nbuf + b
                nxt = (b + 1) % nbuf

                @pl.when(i + 1 < n_ch)
                def _():
                    load(i + 1, nxt)

                pltpu.make_async_copy(
                    idx_hbm.at[pl.ds(0, _CH)], idx_v.at[b], isem.at[b]).wait()
                pltpu.sync_copy(ones_v, acc.at[idx_v.at[b]], add=True)
            return carry

        lax.fori_loop(0, n_ch // nbuf, group, 0)
        plsc.subcore_barrier()

        def wbody(i, carry):
            ch = s * q8 + i
            @pl.when(ch < m8)
            def _():
                pltpu.sync_copy(acc.at[pl.ds(ch * 8, 8)],
                                out.at[pl.ds(row_base + ch * 8, 8)])
            return carry
        lax.fori_loop(0, q8, wbody, 0)

    return k


def _sc_gather(table, idx1, idx2):
    n_idx = idx1.shape[0]
    return _make_sc_gather(n_idx, table.shape[1])(table, idx1, idx2)


def _localize(idx, n_rows):
    half = n_rows // _NC
    rows = []
    for c in range(_NC):
        l = idx - c * half
        rows.append(jnp.where((l >= 0) & (l < half), l, half))
    return jnp.concatenate(rows)


def _sc_scatter(idx1, idx2, vals1, vals2, n_rows):
    w = vals1.shape[1]
    return _make_sc_scatter(idx1.shape[0], n_rows, w)(
        _localize(idx1, n_rows), _localize(idx2, n_rows),
        vals1, vals2, jnp.zeros((8, w), _F32))


def _sc_counts(idx, n_rows):
    return _make_sc_counts(idx.shape[0], n_rows)(
        _localize(idx, n_rows), jnp.ones((_CH, 16), _F32),
        jnp.zeros((8, 16), _F32))


def _row_block(n_rows):
    for r in (640, 512, 320, 160, 80, 8):
        if n_rows % r == 0:
            return r
    return n_rows


def _full(shape):
    return pl.BlockSpec(shape, lambda *_: tuple(0 for _ in shape))


def _tc_embed(x, wee, bee, wm2, bne, wm1, bm):
    be_rows = x.shape[0]
    r = _row_block(be_rows)
    h = wee.shape[1]

    def body(x_ref, wee_ref, bee_ref, wm2_ref, bne_ref, wm1_ref, bm_ref,
             ef_ref, msg_ref):
        ef = _dot(x_ref[...], wee_ref[...]) + bee_ref[...]
        ef_ref[...] = ef
        c0 = _dot(bne_ref[...], wm1_ref[...]) + bm_ref[...]
        msg_ref[...] = jnp.maximum(_dot(ef, wm2_ref[...]) + c0, 0.0)

    return pl.pallas_call(
        body,
        grid=(be_rows // r,),
        in_specs=[pl.BlockSpec((r, x.shape[1]), lambda i: (i, 0)),
                  _full(wee.shape), _full(bee.shape), _full(wm2.shape),
                  _full(bne.shape), _full(wm1.shape), _full(bm.shape)],
        out_specs=[pl.BlockSpec((r, h), lambda i: (i, 0)),
                   pl.BlockSpec((r, h), lambda i: (i, 0))],
        out_shape=[jax.ShapeDtypeStruct((be_rows, h), _F32),
                   jax.ShapeDtypeStruct((be_rows, h), _F32)],
    )(x, wee, bee, wm2, bne, wm1, bm)


def _tc_node_update(agg, counts, prev, s, b):
    bn_rows, h = prev.shape
    r = _row_block(bn_rows)

    def body(agg_ref, cnt_ref, prev_ref, s_ref, b_ref, out_ref):
        cnt = jnp.maximum(cnt_ref[:, 0:1], 1.0)
        out_ref[...] = _ln_rows(agg_ref[...] / cnt + prev_ref[...],
                                s_ref[...], b_ref[...])

    rb = lambda i: (i, 0)
    return pl.pallas_call(
        body,
        grid=(bn_rows // r,),
        in_specs=[pl.BlockSpec((r, h), rb),
                  pl.BlockSpec((r, counts.shape[1]), lambda i: (i, 0)),
                  pl.BlockSpec((r, h), rb), _full(s.shape), _full(b.shape)],
        out_specs=pl.BlockSpec((r, h), rb),
        out_shape=jax.ShapeDtypeStruct((bn_rows, h), _F32),
    )(agg, counts, prev, s, b)


def _tc_edge_block(nr, nc, ef, wn1, wn2, bn, we, be, wc1, wc2, bc, s2, b2,
                   wm1, wm2, bm):
    be_rows, h = ef.shape
    r = _row_block(be_rows)

    def body(nr_ref, nc_ref, ef_ref, wn1_ref, wn2_ref, bn_ref, we_ref,
             be_ref, wc1_ref, wc2_ref, bc_ref, s2_ref, b2_ref,
             wm1_ref, wm2_ref, bm_ref, ef_out, m1_out, m2_out):
        nr_ = nr_ref[...]
        nc_ = nc_ref[...]
        ef_ = ef_ref[...]
        np1 = jnp.maximum(_dot(nr_, wn1_ref[...]) + _dot(nc_, wn2_ref[...])
                          + bn_ref[...], 0.0)
        np2 = jnp.maximum(_dot(nc_, wn1_ref[...]) + _dot(nr_, wn2_ref[...])
                          + bn_ref[...], 0.0)
        ep = jnp.maximum(_dot(ef_, we_ref[...]) + be_ref[...], 0.0)
        sc = _dot(ep, wc2_ref[...]) + bc_ref[...]
        o1 = jnp.maximum(_dot(np1, wc1_ref[...]) + sc, 0.0)
        o2 = jnp.maximum(_dot(np2, wc1_ref[...]) + sc, 0.0)
        efn = _ln_rows((o1 + o2) * 0.5 + ef_, s2_ref[...], b2_ref[...])
        ef_out[...] = efn
        t = _dot(efn, wm2_ref[...]) + bm_ref[...]
        m1_out[...] = jnp.maximum(_dot(nr_, wm1_ref[...]) + t, 0.0)
        m2_out[...] = jnp.maximum(_dot(nc_, wm1_ref[...]) + t, 0.0)

    rb = lambda i: (i, 0)
    w = [wn1, wn2, bn, we, be, wc1, wc2, bc, s2, b2, wm1, wm2, bm]
    return pl.pallas_call(
        body,
        grid=(be_rows // r,),
        in_specs=[pl.BlockSpec((r, h), rb)] * 3 + [_full(a.shape) for a in w],
        out_specs=[pl.BlockSpec((r, h), rb)] * 3,
        out_shape=[jax.ShapeDtypeStruct((be_rows, h), _F32)] * 3,
    )(nr, nc, ef, *w)


def _tc_edge_final(nr, nc, ef, wn1, wn2, bn, we, be, wc1, wc2, bc, s2, b2,
                   wp1, bp1, wp2, bp2, wp3, bp3):
    be_rows, h = ef.shape
    r = _row_block(be_rows)

    def body(nr_ref, nc_ref, ef_ref, wn1_ref, wn2_ref, bn_ref, we_ref,
             be_ref, wc1_ref, wc2_ref, bc_ref, s2_ref, b2_ref,
             wp1_ref, bp1_ref, wp2_ref, bp2_ref, wp3_ref, bp3_ref, out_ref):
        nr_ = nr_ref[...]
        nc_ = nc_ref[...]
        ef_ = ef_ref[...]
        np1 = jnp.maximum(_dot(nr_, wn1_ref[...]) + _dot(nc_, wn2_ref[...])
                          + bn_ref[...], 0.0)
        np2 = jnp.maximum(_dot(nc_, wn1_ref[...]) + _dot(nr_, wn2_ref[...])
                          + bn_ref[...], 0.0)
        ep = jnp.maximum(_dot(ef_, we_ref[...]) + be_ref[...], 0.0)
        sc = _dot(ep, wc2_ref[...]) + bc_ref[...]
        o1 = jnp.maximum(_dot(np1, wc1_ref[...]) + sc, 0.0)
        o2 = jnp.maximum(_dot(np2, wc1_ref[...]) + sc, 0.0)
        efn = _ln_rows((o1 + o2) * 0.5 + ef_, s2_ref[...], b2_ref[...])
        x1 = jnp.maximum(_dot(efn, wp1_ref[...]) + bp1_ref[...], 0.0)
        x2 = jnp.maximum(_dot(x1, wp2_ref[...]) + bp2_ref[...], 0.0)
        out_ref[...] = _dot(x2, wp3_ref[...]) + bp3_ref[...]

    rb = lambda i: (i, 0)
    w = [wn1, wn2, bn, we, be, wc1, wc2, bc, s2, b2,
         wp1, bp1, wp2, bp2, wp3, bp3]
    return pl.pallas_call(
        body,
        grid=(be_rows // r,),
        in_specs=[pl.BlockSpec((r, h), rb)] * 3 + [_full(a.shape) for a in w],
        out_specs=pl.BlockSpec((r, 1), rb),
        out_shape=jax.ShapeDtypeStruct((be_rows, 1), _F32),
    )(nr, nc, ef, *w)


def _tc_softmax(logits3):
    bsz, rows, lanes = logits3.shape

    def body(x_ref, out_ref):
        x = x_ref[...]
        m = jnp.max(x)
        e = jnp.exp(x - m)
        out_ref[...] = e / jnp.sum(e)

    return pl.pallas_call(
        body,
        grid=(bsz,),
        in_specs=[pl.BlockSpec((1, rows, lanes), lambda i: (i, 0, 0))],
        out_specs=pl.BlockSpec((1, rows, lanes), lambda i: (i, 0, 0)),
        out_shape=jax.ShapeDtypeStruct(logits3.shape, _F32),
    )(logits3)


def _tc_value_head(nodes, bsz, n, wv1, bv1, wv2, bv2):
    h = nodes.shape[1]

    def body(nodes_ref, wv1_ref, bv1_ref, wv2_ref, bv2_ref, out_ref):
        g = jnp.mean(nodes_ref[...].reshape(bsz, n, h), axis=1)
        v = jnp.maximum(_dot(g, wv1_ref[...]) + bv1_ref[...], 0.0)
        out_ref[...] = jnp.broadcast_to(
            jnp.tanh(_dot(v, wv2_ref[...]) + bv2_ref[...]), (bsz, 128))

    out = pl.pallas_call(
        body,
        in_specs=[_full(nodes.shape),
                  _full(wv1.shape), _full(bv1.shape),
                  _full(wv2.shape), _full(bv2.shape)],
        out_specs=_full((bsz, 128)),
        out_shape=jax.ShapeDtypeStruct((bsz, 128), _F32),
    )(nodes, wv1, bv1, wv2, bv2)
    return out[:, :1]


def kernel(edge_index, edge_features, params):
    bsz, _, e = edge_index.shape
    n = _N
    be = bsz * e
    bn_rows = bsz * n
    h = params['W_ee'].shape[1]
    nlayers = params['Wm'].shape[0]

    row = edge_index[:, 0, :]
    col = edge_index[:, 1, :]
    offs = (jnp.arange(bsz, dtype=jnp.int32) * n)[:, None]
    rg = (row + offs).reshape(-1)
    cg = (col + offs).reshape(-1)
    idx_all = jnp.concatenate([cg, rg])

    r1 = lambda a: a.reshape(1, -1)

    counts = _sc_counts(idx_all, bn_rows)

    x = jnp.pad(edge_features.reshape(be, edge_features.shape[2]),
                ((0, 0), (0, 5)))
    wee = jnp.pad(params['W_ee'], ((0, 5), (0, 0)))
    ef, msg0 = _tc_embed(x, wee, r1(params['b_ee']),
                         params['Wm'][0][h:], r1(params['b_ne']),
                         params['Wm'][0][:h], r1(params['bm'][0]))

    prev = jnp.broadcast_to(r1(params['b_ne']), (bn_rows, h))
    m1, m2 = msg0, msg0
    nodes = None
    for l in range(nlayers):
        agg = _sc_scatter(cg, rg, m1, m2, bn_rows)
        nodes = _tc_node_update(agg, counts, prev,
                                r1(params['ln1_s'][l]), r1(params['ln1_b'][l]))
        nr, nc = _sc_gather(nodes, rg, cg)
        wn, wc = params['Wn'][l], params['Wc'][l]
        if l < nlayers - 1:
            ef, m1, m2 = _tc_edge_block(
                nr, nc, ef, wn[:h], wn[h:], r1(params['bn'][l]),
                params['We'][l], r1(params['be'][l]),
                wc[:h], wc[h:], r1(params['bc'][l]),
                r1(params['ln2_s'][l]), r1(params['ln2_b'][l]),
                params['Wm'][l + 1][:h], params['Wm'][l + 1][h:],
                r1(params['bm'][l + 1]))
            prev = nodes
        else:
            logits = _tc_edge_final(
                nr, nc, ef, wn[:h], wn[h:], r1(params['bn'][l]),
                params['We'][l], r1(params['be'][l]),
                wc[:h], wc[h:], r1(params['bc'][l]),
                r1(params['ln2_s'][l]), r1(params['ln2_b'][l]),
                params['Wp1'], r1(params['bp1']), params['Wp2'],
                r1(params['bp2']), params['Wp3'], r1(params['bp3']))

    lanes = 128
    policies = _tc_softmax(logits.reshape(bsz, e // lanes, lanes))
    policies = policies.reshape(bsz, e)
    values = _tc_value_head(nodes, bsz, n, params['Wv1'], r1(params['bv1']),
                            params['Wv2'], r1(params['bv2']))
    return policies, values


# speedup vs baseline: 8.9824x; 1.1425x over previous
import functools

import jax
import jax.numpy as jnp
from jax import lax
from jax.experimental import pallas as pl
from jax.experimental.pallas import tpu as pltpu
from jax.experimental.pallas import tpu_sc as plsc

_N = 10000
_NC, _NS = 2, 16
_NW = _NC * _NS
_CH = 80

_F32 = jnp.float32


def _ln_rows(x, s, b):
    m = jnp.mean(x, axis=-1, keepdims=True)
    v = jnp.mean((x - m) ** 2, axis=-1, keepdims=True)
    return (x - m) * lax.rsqrt(v + 1e-6) * s + b


def _dot(a, b):
    return jnp.dot(a, b, preferred_element_type=_F32)


def _nbuf_for(n_ch):
    for nb in (2, 5, 3):
        if n_ch % nb == 0:
            return nb
    return 1


@functools.lru_cache(maxsize=None)
def _make_sc_gather(n_idx, w):
    per_w = n_idx // _NW
    n_ch = per_w // _CH
    assert per_w * _NW == n_idx and n_ch * _CH == per_w
    nbuf = _nbuf_for(n_ch)
    mesh = plsc.VectorSubcoreMesh(core_axis_name="c", subcore_axis_name="s")

    @functools.partial(
        pl.kernel,
        out_type=(jax.ShapeDtypeStruct((n_idx, w), _F32),
                  jax.ShapeDtypeStruct((n_idx, w), _F32)),
        mesh=mesh,
        scratch_types=[pltpu.VMEM((nbuf, _CH), jnp.int32),
                       pltpu.VMEM((nbuf, _CH, w), _F32),
                       pltpu.SemaphoreType.DMA((nbuf,)),
                       pltpu.SemaphoreType.DMA,
                       pltpu.SemaphoreType.DMA((nbuf,))],
        compiler_params=pltpu.CompilerParams(use_tc_tiling_on_sc=False),
    )
    def k(table, idx1, idx2, out1, out2, idx_v, rows_v, isem, gsem, wsem):
        c = lax.axis_index("c")
        s = lax.axis_index("s")
        wid = s * _NC + c
        base0 = wid * per_w

        def make_body(idx_hbm, out_hbm):
            pltpu.async_copy(idx_hbm.at[pl.ds(base0, _CH)],
                             idx_v.at[0], isem.at[0])

            def group(g, carry):
                for b in range(nbuf):
                    i = g * nbuf + b
                    nxt = (b + 1) % nbuf

                    @pl.when(i + 1 < n_ch)
                    def _():
                        pltpu.async_copy(
                            idx_hbm.at[pl.ds(base0 + (i + 1) * _CH, _CH)],
                            idx_v.at[nxt], isem.at[nxt])

                    pltpu.make_async_copy(
                        idx_hbm.at[pl.ds(base0, _CH)], idx_v.at[b],
                        isem.at[b]).wait()

                    @pl.when(i >= nbuf)
                    def _():
                        pltpu.make_async_copy(
                            rows_v.at[b], out_hbm.at[pl.ds(base0, _CH)],
                            wsem.at[b]).wait()

                    pltpu.async_copy(table.at[idx_v.at[b]], rows_v.at[b],
                                     gsem).wait()
                    pltpu.async_copy(rows_v.at[b],
                                     out_hbm.at[pl.ds(base0 + i * _CH, _CH)],
                                     wsem.at[b])
                return carry

            lax.fori_loop(0, n_ch // nbuf, group, 0)
            for b in range(min(nbuf, n_ch)):
                pltpu.make_async_copy(
                    rows_v.at[b], out_hbm.at[pl.ds(base0, _CH)],
                    wsem.at[b]).wait()

        make_body(idx1, out1)
        make_body(idx2, out2)

    return k


@functools.lru_cache(maxsize=None)
def _make_sc_scatter(n_idx, n_rows, w):
    n_ch = n_idx // _CH // _NS
    assert n_ch * _CH * _NS == n_idx
    assert n_rows % _CH == 0
    m80 = n_rows // _CH
    q80 = -(-m80 // _NS)
    nbuf = _nbuf_for(n_ch)
    mesh = plsc.VectorSubcoreMesh(core_axis_name="c", subcore_axis_name="s",
                                  num_cores=1)

    @functools.partial(
        pl.kernel,
        out_type=jax.ShapeDtypeStruct((n_rows, w), _F32),
        mesh=mesh,
        scratch_types=[pltpu.VMEM((nbuf, _CH), jnp.int32),
                       pltpu.VMEM((nbuf, _CH, w), _F32),
                       pltpu.VMEM((_CH, w), _F32),
                       pltpu.VMEM_SHARED((n_rows, w), _F32),
                       pltpu.SemaphoreType.DMA((nbuf,)),
                       pltpu.SemaphoreType.DMA((nbuf,))],
        compiler_params=pltpu.CompilerParams(use_tc_tiling_on_sc=False),
    )
    def k(idx1, idx2, vals1, vals2, zeros_hbm, out,
          idx_v, val_v, z_v, acc, isem, vsem):
        s = lax.axis_index("s")

        pltpu.sync_copy(zeros_hbm, z_v)

        def zbody(i, carry):
            ch = s * q80 + i
            @pl.when(ch < m80)
            def _():
                pltpu.sync_copy(z_v, acc.at[pl.ds(ch * _CH, _CH)])
            return carry
        lax.fori_loop(0, q80, zbody, 0)
        plsc.subcore_barrier()

        def make_body(idx_hbm, vals_hbm):
            def load(i, b):
                base = (s * n_ch + i) * _CH
                pltpu.async_copy(idx_hbm.at[pl.ds(base, _CH)],
                                 idx_v.at[b], isem.at[b])
                pltpu.async_copy(vals_hbm.at[pl.ds(base, _CH)],
                                 val_v.at[b], vsem.at[b])

            load(0, 0)

            def group(g, carry):
                for b in range(nbuf):
                    i = g * nbuf + b
                    nxt = (b + 1) % nbuf

                    @pl.when(i + 1 < n_ch)
                    def _():
                        load(i + 1, nxt)

                    pltpu.make_async_copy(
                        idx_hbm.at[pl.ds(0, _CH)], idx_v.at[b],
                        isem.at[b]).wait()
                    pltpu.make_async_copy(
                        vals_hbm.at[pl.ds(0, _CH)], val_v.at[b],
                        vsem.at[b]).wait()
                    pltpu.sync_copy(val_v.at[b], acc.at[idx_v.at[b]],
                                    add=True)
                return carry

            lax.fori_loop(0, n_ch // nbuf, group, 0)

        make_body(idx1, vals1)
        make_body(idx2, vals2)
        plsc.subcore_barrier()

        def wbody(i, carry):
            ch = s * q80 + i
            @pl.when(ch < m80)
            def _():
                pltpu.sync_copy(acc.at[pl.ds(ch * _CH, _CH)],
                                out.at[pl.ds(ch * _CH, _CH)])
            return carry
        lax.fori_loop(0, q80, wbody, 0)

    return k


@functools.lru_cache(maxsize=None)
def _make_sc_counts(n_idx, n_rows):
    w = 16
    n_ch = n_idx // _CH // _NS
    assert n_ch * _CH * _NS == n_idx
    assert n_rows % _CH == 0
    m80 = n_rows // _CH
    q80 = -(-m80 // _NS)
    nbuf = _nbuf_for(n_ch)
    mesh = plsc.VectorSubcoreMesh(core_axis_name="c", subcore_axis_name="s",
                                  num_cores=1)

    @functools.partial(
        pl.kernel,
        out_type=jax.ShapeDtypeStruct((n_rows, w), _F32),
        mesh=mesh,
        scratch_types=[pltpu.VMEM((nbuf, _CH), jnp.int32),
                       pltpu.VMEM((_CH, w), _F32),
                       pltpu.VMEM((_CH, w), _F32),
                       pltpu.VMEM_SHARED((n_rows, w), _F32),
                       pltpu.SemaphoreType.DMA((nbuf,))],
        compiler_params=pltpu.CompilerParams(use_tc_tiling_on_sc=False),
    )
    def k(idx_hbm, ones_hbm, zeros_hbm, out, idx_v, ones_v, z_v, acc, isem):
        s = lax.axis_index("s")

        pltpu.sync_copy(zeros_hbm, z_v)
        pltpu.sync_copy(ones_hbm, ones_v)

        def zbody(i, carry):
            ch = s * q80 + i
            @pl.when(ch < m80)
            def _():
                pltpu.sync_copy(z_v, acc.at[pl.ds(ch * _CH, _CH)])
            return carry
        lax.fori_loop(0, q80, zbody, 0)
        plsc.subcore_barrier()

        def load(i, b):
            base = (s * n_ch + i) * _CH
            pltpu.async_copy(idx_hbm.at[pl.ds(base, _CH)],
                             idx_v.at[b], isem.at[b])

        load(0, 0)

        def group(g, carry):
            for b in range(nbuf):
                i = g * nbuf + b
                nxt = (b + 1) % nbuf

                @pl.when(i + 1 < n_ch)
                def _():
                    load(i + 1, nxt)

                pltpu.make_async_copy(
                    idx_hbm.at[pl.ds(0, _CH)], idx_v.at[b], isem.at[b]).wait()
                pltpu.sync_copy(ones_v, acc.at[idx_v.at[b]], add=True)
            return carry

        lax.fori_loop(0, n_ch // nbuf, group, 0)
        plsc.subcore_barrier()

        def wbody(i, carry):
            ch = s * q80 + i
            @pl.when(ch < m80)
            def _():
                pltpu.sync_copy(acc.at[pl.ds(ch * _CH, _CH)],
                                out.at[pl.ds(ch * _CH, _CH)])
            return carry
        lax.fori_loop(0, q80, wbody, 0)

    return k


def _sc_gather(table, idx1, idx2):
    n_idx = idx1.shape[0]
    return _make_sc_gather(n_idx, table.shape[1])(table, idx1, idx2)


def _sc_scatter(idx1, idx2, vals1, vals2, n_rows):
    w = vals1.shape[1]
    return _make_sc_scatter(idx1.shape[0], n_rows, w)(
        idx1, idx2, vals1, vals2, jnp.zeros((_CH, w), _F32))


def _sc_counts(idx, n_rows):
    return _make_sc_counts(idx.shape[0], n_rows)(
        idx, jnp.ones((_CH, 16), _F32), jnp.zeros((_CH, 16), _F32))


def _row_block(n_rows):
    for r in (640, 512, 320, 160, 80, 8):
        if n_rows % r == 0:
            return r
    return n_rows


def _full(shape):
    return pl.BlockSpec(shape, lambda *_: tuple(0 for _ in shape))


def _tc_embed(x, wee, bee, wm2, bne, wm1, bm):
    be_rows = x.shape[0]
    r = _row_block(be_rows)
    h = wee.shape[1]

    def body(x_ref, wee_ref, bee_ref, wm2_ref, bne_ref, wm1_ref, bm_ref,
             ef_ref, msg_ref):
        ef = _dot(x_ref[...], wee_ref[...]) + bee_ref[...]
        ef_ref[...] = ef
        c0 = _dot(bne_ref[...], wm1_ref[...]) + bm_ref[...]
        msg_ref[...] = jnp.maximum(_dot(ef, wm2_ref[...]) + c0, 0.0)

    return pl.pallas_call(
        body,
        grid=(be_rows // r,),
        in_specs=[pl.BlockSpec((r, x.shape[1]), lambda i: (i, 0)),
                  _full(wee.shape), _full(bee.shape), _full(wm2.shape),
                  _full(bne.shape), _full(wm1.shape), _full(bm.shape)],
        out_specs=[pl.BlockSpec((r, h), lambda i: (i, 0)),
                   pl.BlockSpec((r, h), lambda i: (i, 0))],
        out_shape=[jax.ShapeDtypeStruct((be_rows, h), _F32),
                   jax.ShapeDtypeStruct((be_rows, h), _F32)],
    )(x, wee, bee, wm2, bne, wm1, bm)


def _tc_node_update(agg, counts, prev, s, b):
    bn_rows, h = prev.shape
    r = _row_block(bn_rows)

    def body(agg_ref, cnt_ref, prev_ref, s_ref, b_ref, out_ref):
        cnt = jnp.maximum(cnt_ref[:, 0:1], 1.0)
        out_ref[...] = _ln_rows(agg_ref[...] / cnt + prev_ref[...],
                                s_ref[...], b_ref[...])

    rb = lambda i: (i, 0)
    return pl.pallas_call(
        body,
        grid=(bn_rows // r,),
        in_specs=[pl.BlockSpec((r, h), rb),
                  pl.BlockSpec((r, counts.shape[1]), lambda i: (i, 0)),
                  pl.BlockSpec((r, h), rb), _full(s.shape), _full(b.shape)],
        out_specs=pl.BlockSpec((r, h), rb),
        out_shape=jax.ShapeDtypeStruct((bn_rows, h), _F32),
    )(agg, counts, prev, s, b)


def _tc_edge_block(nr, nc, ef, wn1, wn2, bn, we, be, wc1, wc2, bc, s2, b2,
                   wm1, wm2, bm):
    be_rows, h = ef.shape
    r = _row_block(be_rows)

    def body(nr_ref, nc_ref, ef_ref, wn1_ref, wn2_ref, bn_ref, we_ref,
             be_ref, wc1_ref, wc2_ref, bc_ref, s2_ref, b2_ref,
             wm1_ref, wm2_ref, bm_ref, ef_out, m1_out, m2_out):
        nr_ = nr_ref[...]
        nc_ = nc_ref[...]
        ef_ = ef_ref[...]
        np1 = jnp.maximum(_dot(nr_, wn1_ref[...]) + _dot(nc_, wn2_ref[...])
                          + bn_ref[...], 0.0)
        np2 = jnp.maximum(_dot(nc_, wn1_ref[...]) + _dot(nr_, wn2_ref[...])
                          + bn_ref[...], 0.0)
        ep = jnp.maximum(_dot(ef_, we_ref[...]) + be_ref[...], 0.0)
        sc = _dot(ep, wc2_ref[...]) + bc_ref[...]
        o1 = jnp.maximum(_dot(np1, wc1_ref[...]) + sc, 0.0)
        o2 = jnp.maximum(_dot(np2, wc1_ref[...]) + sc, 0.0)
        efn = _ln_rows((o1 + o2) * 0.5 + ef_, s2_ref[...], b2_ref[...])
        ef_out[...] = efn
        t = _dot(efn, wm2_ref[...]) + bm_ref[...]
        m1_out[...] = jnp.maximum(_dot(nr_, wm1_ref[...]) + t, 0.0)
        m2_out[...] = jnp.maximum(_dot(nc_, wm1_ref[...]) + t, 0.0)

    rb = lambda i: (i, 0)
    w = [wn1, wn2, bn, we, be, wc1, wc2, bc, s2, b2, wm1, wm2, bm]
    return pl.pallas_call(
        body,
        grid=(be_rows // r,),
        in_specs=[pl.BlockSpec((r, h), rb)] * 3 + [_full(a.shape) for a in w],
        out_specs=[pl.BlockSpec((r, h), rb)] * 3,
        out_shape=[jax.ShapeDtypeStruct((be_rows, h), _F32)] * 3,
    )(nr, nc, ef, *w)


def _tc_edge_final(nr, nc, ef, wn1, wn2, bn, we, be, wc1, wc2, bc, s2, b2,
                   wp1, bp1, wp2, bp2, wp3, bp3):
    be_rows, h = ef.shape
    r = _row_block(be_rows)

    def body(nr_ref, nc_ref, ef_ref, wn1_ref, wn2_ref, bn_ref, we_ref,
             be_ref, wc1_ref, wc2_ref, bc_ref, s2_ref, b2_ref,
             wp1_ref, bp1_ref, wp2_ref, bp2_ref, wp3_ref, bp3_ref, out_ref):
        nr_ = nr_ref[...]
        nc_ = nc_ref[...]
        ef_ = ef_ref[...]
        np1 = jnp.maximum(_dot(nr_, wn1_ref[...]) + _dot(nc_, wn2_ref[...])
                          + bn_ref[...], 0.0)
        np2 = jnp.maximum(_dot(nc_, wn1_ref[...]) + _dot(nr_, wn2_ref[...])
                          + bn_ref[...], 0.0)
        ep = jnp.maximum(_dot(ef_, we_ref[...]) + be_ref[...], 0.0)
        sc = _dot(ep, wc2_ref[...]) + bc_ref[...]
        o1 = jnp.maximum(_dot(np1, wc1_ref[...]) + sc, 0.0)
        o2 = jnp.maximum(_dot(np2, wc1_ref[...]) + sc, 0.0)
        efn = _ln_rows((o1 + o2) * 0.5 + ef_, s2_ref[...], b2_ref[...])
        x1 = jnp.maximum(_dot(efn, wp1_ref[...]) + bp1_ref[...], 0.0)
        x2 = jnp.maximum(_dot(x1, wp2_ref[...]) + bp2_ref[...], 0.0)
        out_ref[...] = _dot(x2, wp3_ref[...]) + bp3_ref[...]

    rb = lambda i: (i, 0)
    w = [wn1, wn2, bn, we, be, wc1, wc2, bc, s2, b2,
         wp1, bp1, wp2, bp2, wp3, bp3]
    return pl.pallas_call(
        body,
        grid=(be_rows // r,),
        in_specs=[pl.BlockSpec((r, h), rb)] * 3 + [_full(a.shape) for a in w],
        out_specs=pl.BlockSpec((r, 1), rb),
        out_shape=jax.ShapeDtypeStruct((be_rows, 1), _F32),
    )(nr, nc, ef, *w)


def _tc_softmax(logits3):
    bsz, rows, lanes = logits3.shape

    def body(x_ref, out_ref):
        x = x_ref[...]
        m = jnp.max(x)
        e = jnp.exp(x - m)
        out_ref[...] = e / jnp.sum(e)

    return pl.pallas_call(
        body,
        grid=(bsz,),
        in_specs=[pl.BlockSpec((1, rows, lanes), lambda i: (i, 0, 0))],
        out_specs=pl.BlockSpec((1, rows, lanes), lambda i: (i, 0, 0)),
        out_shape=jax.ShapeDtypeStruct(logits3.shape, _F32),
    )(logits3)


def _tc_value_head(nodes, bsz, n, wv1, bv1, wv2, bv2):
    h = nodes.shape[1]

    def body(nodes_ref, wv1_ref, bv1_ref, wv2_ref, bv2_ref, out_ref):
        g = jnp.mean(nodes_ref[...].reshape(bsz, n, h), axis=1)
        v = jnp.maximum(_dot(g, wv1_ref[...]) + bv1_ref[...], 0.0)
        out_ref[...] = jnp.broadcast_to(
            jnp.tanh(_dot(v, wv2_ref[...]) + bv2_ref[...]), (bsz, 128))

    out = pl.pallas_call(
        body,
        in_specs=[_full(nodes.shape),
                  _full(wv1.shape), _full(bv1.shape),
                  _full(wv2.shape), _full(bv2.shape)],
        out_specs=_full((bsz, 128)),
        out_shape=jax.ShapeDtypeStruct((bsz, 128), _F32),
    )(nodes, wv1, bv1, wv2, bv2)
    return out[:, :1]


def kernel(edge_index, edge_features, params):
    bsz, _, e = edge_index.shape
    n = _N
    be = bsz * e
    bn_rows = bsz * n
    h = params['W_ee'].shape[1]
    nlayers = params['Wm'].shape[0]

    row = edge_index[:, 0, :]
    col = edge_index[:, 1, :]
    offs = (jnp.arange(bsz, dtype=jnp.int32) * n)[:, None]
    rg = (row + offs).reshape(-1)
    cg = (col + offs).reshape(-1)
    idx_all = jnp.concatenate([cg, rg])

    r1 = lambda a: a.reshape(1, -1)

    counts = _sc_counts(idx_all, bn_rows)

    x = jnp.pad(edge_features.reshape(be, edge_features.shape[2]),
                ((0, 0), (0, 5)))
    wee = jnp.pad(params['W_ee'], ((0, 5), (0, 0)))
    ef, msg0 = _tc_embed(x, wee, r1(params['b_ee']),
                         params['Wm'][0][h:], r1(params['b_ne']),
                         params['Wm'][0][:h], r1(params['bm'][0]))

    prev = jnp.broadcast_to(r1(params['b_ne']), (bn_rows, h))
    m1, m2 = msg0, msg0
    nodes = None
    for l in range(nlayers):
        agg = _sc_scatter(cg, rg, m1, m2, bn_rows)
        nodes = _tc_node_update(agg, counts, prev,
                                r1(params['ln1_s'][l]), r1(params['ln1_b'][l]))
        nr, nc = _sc_gather(nodes, rg, cg)
        wn, wc = params['Wn'][l], params['Wc'][l]
        if l < nlayers - 1:
            ef, m1, m2 = _tc_edge_block(
                nr, nc, ef, wn[:h], wn[h:], r1(params['bn'][l]),
                params['We'][l], r1(params['be'][l]),
                wc[:h], wc[h:], r1(params['bc'][l]),
                r1(params['ln2_s'][l]), r1(params['ln2_b'][l]),
                params['Wm'][l + 1][:h], params['Wm'][l + 1][h:],
                r1(params['bm'][l + 1]))
            prev = nodes
        else:
            logits = _tc_edge_final(
                nr, nc, ef, wn[:h], wn[h:], r1(params['bn'][l]),
                params['We'][l], r1(params['be'][l]),
                wc[:h], wc[h:], r1(params['bc'][l]),
                r1(params['ln2_s'][l]), r1(params['ln2_b'][l]),
                params['Wp1'], r1(params['bp1']), params['Wp2'],
                r1(params['bp2']), params['Wp3'], r1(params['bp3']))

    lanes = 128
    policies = _tc_softmax(logits.reshape(bsz, e // lanes, lanes))
    policies = policies.reshape(bsz, e)
    values = _tc_value_head(nodes, bsz, n, params['Wv1'], r1(params['bv1']),
                            params['Wv2'], r1(params['bv2']))
    return policies, values
